# baseline (speedup 1.0000x reference)
import functools

import jax
import jax.numpy as jnp
from jax import lax
from jax.experimental import pallas as pl
from jax.experimental.pallas import tpu as pltpu
from jax.experimental.pallas import tpu_sc as plsc

CTX = 15
VOCAB = 100000
EMB = 128
HID = 1024
BATCH = 1024

NTOK = BATCH * CTX
NUM_WORKERS = 32
TOK_PER_W = NTOK // NUM_WORKERS

VBLK = 2048
NVB = (VOCAB + VBLK - 1) // VBLK


_sc_mesh = plsc.VectorSubcoreMesh(core_axis_name="c", subcore_axis_name="s")


@functools.partial(
    pl.kernel,
    out_type=jax.ShapeDtypeStruct((NTOK, EMB), jnp.float32),
    mesh=_sc_mesh,
    scratch_types=[
        pltpu.VMEM((TOK_PER_W,), jnp.int32),
        pltpu.VMEM((TOK_PER_W, EMB), jnp.float32),
        pltpu.SemaphoreType.DMA,
    ],
)
def _gather_rows(emb_hbm, idx_hbm, out_hbm, idx_v, rows_v, sem):
    wid = lax.axis_index("s") * 2 + lax.axis_index("c")
    base = wid * TOK_PER_W
    pltpu.sync_copy(idx_hbm.at[pl.ds(base, TOK_PER_W)], idx_v)
    pltpu.async_copy(emb_hbm.at[idx_v], rows_v, sem).wait()
    pltpu.sync_copy(rows_v, out_hbm.at[pl.ds(base, TOK_PER_W)])


def _mlp_body(e_ref, w1_ref, b1_ref, w2_ref, b2_ref, out_ref, h_ref):
    @pl.when(pl.program_id(0) == 0)
    def _():
        e = e_ref[...].astype(jnp.bfloat16)
        w1 = w1_ref[...].astype(jnp.bfloat16)
        h = jnp.dot(e, w1, preferred_element_type=jnp.float32)
        h_ref[...] = jnp.maximum(h + b1_ref[...], 0.0).astype(jnp.bfloat16)

    w2 = w2_ref[...].astype(jnp.bfloat16)
    acc = jnp.dot(h_ref[...], w2, preferred_element_type=jnp.float32)
    out_ref[...] = acc + b2_ref[...]


_mlp = pl.pallas_call(
    _mlp_body,
    grid=(NVB,),
    in_specs=[
        pl.BlockSpec((BATCH, CTX * EMB), lambda j: (0, 0)),
        pl.BlockSpec((CTX * EMB, HID), lambda j: (0, 0)),
        pl.BlockSpec((1, HID), lambda j: (0, 0)),
        pl.BlockSpec((HID, VBLK), lambda j: (0, j)),
        pl.BlockSpec((1, VBLK), lambda j: (0, j)),
    ],
    out_specs=pl.BlockSpec((BATCH, VBLK), lambda j: (0, j)),
    out_shape=jax.ShapeDtypeStruct((BATCH, VOCAB), jnp.float32),
    scratch_shapes=[pltpu.VMEM((BATCH, HID), jnp.bfloat16)],
    compiler_params=pltpu.CompilerParams(
        dimension_semantics=("arbitrary",),
    ),
)


def kernel(x, emb, W1, b1, W2, b2):
    idx = x.reshape(-1).astype(jnp.int32)
    rows = _gather_rows(emb, idx)
    e_flat = rows.reshape(BATCH, CTX * EMB)
    return _mlp(e_flat, W1, b1.reshape(1, HID), W2, b2.reshape(1, VOCAB))
